# Initial kernel scaffold; baseline (speedup 1.0000x reference)
import jax
import jax.numpy as jnp
from jax.experimental import pallas as pl


def kernel(x, edge_index, adj_vals, kernel, kernel1, kernel2):
    raise NotImplementedError("write your pallas kernel here")


# speedup vs baseline: 16.6499x; 16.6499x over previous
import functools

import jax
import jax.numpy as jnp
from jax import lax
from jax.experimental import pallas as pl
from jax.experimental.pallas import tpu as pltpu
from jax.experimental.pallas import tpu_sc as plsc

N = 10000
E = 320000
D = 128
OUT = 128
ROWW = 144

NC = 2
NS = 16
NW = NC * NS
EPT = E // NW
K = 80
NCHUNK = EPT // K
KG = K // 16


def _tc_prep_body(x_ref, k_ref, k1_ref, k2_ref, mp_ref, c1_ref, c2_ref):
    x = x_ref[...]
    mean = jnp.mean(x, axis=0, keepdims=True)
    xc = x - mean
    var = jnp.mean(xc * xc, axis=0, keepdims=True)
    xb = xc / jnp.sqrt(var + 1e-3)
    mapped = jnp.dot(xb, k_ref[...], preferred_element_type=jnp.float32)
    a1 = jnp.dot(xb, k1_ref[...], preferred_element_type=jnp.float32)
    a2 = jnp.dot(xb, k2_ref[...], preferred_element_type=jnp.float32)
    c1_ref[...] = jnp.tanh(jnp.sum(a1 * xb, axis=1, keepdims=True))
    c2_ref[...] = jnp.tanh(jnp.sum(a2 * xb, axis=1, keepdims=True))
    col = lax.broadcasted_iota(jnp.int32, (1, ROWW - OUT), 1)
    pad = jnp.where(col == 0, 1.0, 0.0).astype(jnp.float32)
    pad = jnp.broadcast_to(pad, (N, ROWW - OUT))
    mp_ref[...] = jnp.concatenate([mapped, pad], axis=1)


_tc_prep = pl.pallas_call(
    _tc_prep_body,
    out_shape=(
        jax.ShapeDtypeStruct((N, ROWW), jnp.float32),
        jax.ShapeDtypeStruct((N, 1), jnp.float32),
        jax.ShapeDtypeStruct((N, 1), jnp.float32),
    ),
)


def _sc_edge_body(mp_hbm, rows_hbm, cols_hbm, adj_hbm, c1_hbm, c2_hbm, out_hbm,
                  c1_v, c2_v, buf, ridx, cidx, adj_c, ev_v, acc_sh, sem):
    cid = lax.axis_index("c")
    sid = lax.axis_index("s")
    wid = cid * NS + sid
    ebase = wid * EPT

    pltpu.sync_copy(c1_hbm, c1_v)
    pltpu.sync_copy(c2_hbm, c2_v)

    zero16 = jnp.zeros((16,), jnp.float32)

    def _zb(j, carry):
        for k9 in range(ROWW // 16):
            buf[j, pl.ds(k9 * 16, 16)] = zero16
        return carry

    lax.fori_loop(0, K, _zb, 0)

    for i in range(NCHUNK):
        @pl.when(sid == (i % NS))
        def _():
            pltpu.sync_copy(buf, acc_sh.at[pl.ds(i * K, K)])

    plsc.subcore_barrier()

    def _chunk(i, carry):
        base = ebase + i * K
        pltpu.sync_copy(rows_hbm.at[pl.ds(base, K)], ridx)
        pltpu.sync_copy(cols_hbm.at[pl.ds(base, K)], cidx)
        pltpu.sync_copy(adj_hbm.at[pl.ds(base, K)], adj_c)
        pltpu.async_copy(mp_hbm.at[cidx], buf, sem).wait()
        for g in range(KG):
            r16 = ridx[pl.ds(g * 16, 16)]
            c16 = cidx[pl.ds(g * 16, 16)]
            a16 = adj_c[pl.ds(g * 16, 16)]
            v = plsc.load_gather(c1_v, [r16])
            w = plsc.load_gather(c2_v, [c16])
            val = a16 * (v + w)
            val = jnp.maximum(val, 0.2 * val)
            ev_v[pl.ds(g * 16, 16)] = jnp.exp(val)
        def _scale(g, c2_):
            ev16 = ev_v[pl.ds(g * 16, 16)]
            for jj in range(16):
                j = g * 16 + jj
                e = ev16[jj]
                for k9 in range(ROWW // 16):
                    buf[j, pl.ds(k9 * 16, 16)] = buf[j, pl.ds(k9 * 16, 16)] * e
            return c2_

        lax.fori_loop(0, KG, _scale, 0)
        pltpu.sync_copy(buf, acc_sh.at[ridx], add=True)
        return carry

    lax.fori_loop(0, NCHUNK, _chunk, 0)

    plsc.subcore_barrier()

    @pl.when(sid == 0)
    def _():
        pltpu.sync_copy(acc_sh, out_hbm.at[cid])


_sc_edge = functools.partial(
    pl.kernel,
    mesh=plsc.VectorSubcoreMesh(core_axis_name="c", subcore_axis_name="s"),
    out_type=jax.ShapeDtypeStruct((NC, N, ROWW), jnp.float32),
    compiler_params=pltpu.CompilerParams(
        needs_layout_passes=False, use_tc_tiling_on_sc=False),
    scratch_types=[
        pltpu.VMEM((N,), jnp.float32),
        pltpu.VMEM((N,), jnp.float32),
        pltpu.VMEM((K, ROWW), jnp.float32),
        pltpu.VMEM((K,), jnp.int32),
        pltpu.VMEM((K,), jnp.int32),
        pltpu.VMEM((K,), jnp.float32),
        pltpu.VMEM((K,), jnp.float32),
        pltpu.VMEM_SHARED((N, ROWW), jnp.float32),
        pltpu.SemaphoreType.DMA,
    ],
)(_sc_edge_body)


def _tc_final_body(acc_ref, o_ref):
    a = acc_ref[0] + acc_ref[1]
    col = lax.broadcasted_iota(jnp.int32, (1, ROWW), 1)
    dmask = (col == OUT).astype(jnp.float32)
    denom = jnp.sum(a * dmask, axis=1, keepdims=True)
    s = a[:, 0:OUT]
    o_ref[...] = jnp.tanh(s / jnp.where(denom > 0, denom, 1.0))


_tc_final = pl.pallas_call(
    _tc_final_body,
    out_shape=jax.ShapeDtypeStruct((N, OUT), jnp.float32),
)


def kernel(x, edge_index, adj_vals, kernel, kernel1, kernel2):
    mappedP, c1, c2 = _tc_prep(x, kernel, kernel1, kernel2)
    c1 = c1.reshape(N)
    c2 = c2.reshape(N)
    rows = edge_index[0]
    cols = edge_index[1]
    acc = _sc_edge(mappedP, rows, cols, adj_vals, c1, c2)
    return _tc_final(acc)


# speedup vs baseline: 31.6211x; 1.8992x over previous
import functools

import jax
import jax.numpy as jnp
from jax import lax
from jax.experimental import pallas as pl
from jax.experimental.pallas import tpu as pltpu
from jax.experimental.pallas import tpu_sc as plsc

N = 10000
E = 320000
D = 128
OUT = 128
ROWW = 144

NC = 2
NS = 16
NW = NC * NS
EPT = E // NW
K = 64
NCHUNK = EPT // K
TAIL = EPT - NCHUNK * K
KG = K // 16
LAST = NCHUNK - 1


def _tc_prep_body(x_ref, k_ref, k1_ref, k2_ref, mp_ref, c1_ref, c2_ref):
    x = x_ref[...]
    mean = jnp.mean(x, axis=0, keepdims=True)
    xc = x - mean
    var = jnp.mean(xc * xc, axis=0, keepdims=True)
    xb = xc / jnp.sqrt(var + 1e-3)
    mapped = jnp.dot(xb, k_ref[...], preferred_element_type=jnp.float32)
    a1 = jnp.dot(xb, k1_ref[...], preferred_element_type=jnp.float32)
    a2 = jnp.dot(xb, k2_ref[...], preferred_element_type=jnp.float32)
    c1_ref[...] = jnp.tanh(jnp.sum(a1 * xb, axis=1, keepdims=True))
    c2_ref[...] = jnp.tanh(jnp.sum(a2 * xb, axis=1, keepdims=True))
    col = lax.broadcasted_iota(jnp.int32, (1, ROWW - OUT), 1)
    pad = jnp.where(col == 0, 1.0, 0.0).astype(jnp.float32)
    pad = jnp.broadcast_to(pad, (N, ROWW - OUT))
    mp_ref[...] = jnp.concatenate([mapped, pad], axis=1)


_tc_prep = pl.pallas_call(
    _tc_prep_body,
    out_shape=(
        jax.ShapeDtypeStruct((N, ROWW), jnp.float32),
        jax.ShapeDtypeStruct((N, 1), jnp.float32),
        jax.ShapeDtypeStruct((N, 1), jnp.float32),
    ),
)


def _sc_edge_body(mp_hbm, e3_hbm, c1_hbm, c2_hbm, out_hbm,
                  c1_v, c2_v, buf0, buf1, e3_0, e3_1, rb0, rb1, tb, ev_v,
                  acc_sh, gsem, ssem, isem):
    cid = lax.axis_index("c")
    sid = lax.axis_index("s")
    wid = cid * NS + sid
    ebase = wid * EPT
    bufs = (buf0, buf1)
    e3s = (e3_0, e3_1)
    rbs = (rb0, rb1)

    pltpu.sync_copy(c1_hbm, c1_v)
    pltpu.sync_copy(c2_hbm, c2_v)

    zero16 = jnp.zeros((16,), jnp.float32)

    def _zb(j, carry):
        for k9 in range(ROWW // 16):
            buf0[j, pl.ds(k9 * 16, 16)] = zero16
            buf1[j, pl.ds(k9 * 16, 16)] = zero16
        return carry

    lax.fori_loop(0, K, _zb, 0)

    for i in range(NCHUNK):
        @pl.when(sid == (i % NS))
        def _():
            pltpu.sync_copy(buf0, acc_sh.at[pl.ds(i * K, K)])

    @pl.when(sid == 0)
    def _():
        pltpu.sync_copy(buf0.at[pl.ds(0, TAIL)],
                        acc_sh.at[pl.ds(NCHUNK * K, TAIL)])

    plsc.subcore_barrier()

    def _snap_rows(e3b, rb):
        for g in range(KG):
            rb[pl.ds(g * 16, 16)] = e3b[0, pl.ds(g * 16, 16)]

    def _compute_ev(e3b):
        for g in range(KG):
            r16 = e3b[0, pl.ds(g * 16, 16)]
            c16 = e3b[1, pl.ds(g * 16, 16)]
            a16 = plsc.bitcast(e3b[2, pl.ds(g * 16, 16)], jnp.float32)
            v = plsc.load_gather(c1_v, [r16])
            w = plsc.load_gather(c2_v, [c16])
            val = a16 * (v + w)
            val = jnp.maximum(val, 0.2 * val)
            ev_v[pl.ds(g * 16, 16)] = jnp.exp(val)

    def _scale_buf(bufb):
        def _scale(g, carry):
            ev16 = ev_v[pl.ds(g * 16, 16)]
            for jj in range(16):
                j = g * 16 + jj
                e = ev16[jj]
                for k9 in range(ROWW // 16):
                    bufb[j, pl.ds(k9 * 16, 16)] = \
                        bufb[j, pl.ds(k9 * 16, 16)] * e
            return carry

        lax.fori_loop(0, KG, _scale, 0)

    pltpu.sync_copy(e3_hbm.at[:, pl.ds(ebase, K)], e3_0)
    _snap_rows(e3_0, rb1)
    pltpu.async_copy(mp_hbm.at[e3_0.at[1]], buf0, gsem)
    pltpu.async_copy(e3_hbm.at[:, pl.ds(ebase + K, K)], e3_1, isem)
    pltpu.async_copy(buf1, acc_sh.at[rb1], ssem, add=True)

    def _body(i, b):
        bo = 1 - b
        base = ebase + i * K
        pltpu.make_async_copy(mp_hbm.at[e3s[b].at[1]], bufs[b], gsem).wait()
        pltpu.make_async_copy(bufs[bo], acc_sh.at[rbs[bo]], ssem).wait()

        @pl.when(i <= LAST - 1)
        def _():
            pltpu.make_async_copy(
                e3_hbm.at[:, pl.ds(base + K, K)], e3s[bo], isem).wait()
            pltpu.async_copy(mp_hbm.at[e3s[bo].at[1]], bufs[bo], gsem)

        _snap_rows(e3s[b], rbs[b])
        _compute_ev(e3s[b])

        @pl.when(i <= LAST - 2)
        def _():
            pltpu.async_copy(
                e3_hbm.at[:, pl.ds(base + 2 * K, K)], e3s[b], isem)

        _scale_buf(bufs[b])
        pltpu.async_copy(bufs[b], acc_sh.at[rbs[b]], ssem, add=True)

    def _pair(p, carry):
        _body(2 * p, 0)
        _body(2 * p + 1, 1)
        return carry

    lax.fori_loop(0, NCHUNK // 2, _pair, 0)

    pltpu.make_async_copy(bufs[1], acc_sh.at[rbs[1]], ssem).wait()

    tbase = ebase + NCHUNK * K
    pltpu.sync_copy(e3_hbm.at[:, pl.ds(tbase, TAIL)], tb)
    pltpu.async_copy(mp_hbm.at[tb.at[1]], buf0.at[pl.ds(0, TAIL)], gsem).wait()
    r16 = tb[0, pl.ds(0, 16)]
    c16 = tb[1, pl.ds(0, 16)]
    a16 = plsc.bitcast(tb[2, pl.ds(0, 16)], jnp.float32)
    v = plsc.load_gather(c1_v, [r16])
    w = plsc.load_gather(c2_v, [c16])
    val = a16 * (v + w)
    val = jnp.maximum(val, 0.2 * val)
    ev16 = jnp.exp(val)
    for jj in range(TAIL):
        e = ev16[jj]
        for k9 in range(ROWW // 16):
            buf0[jj, pl.ds(k9 * 16, 16)] = buf0[jj, pl.ds(k9 * 16, 16)] * e
    pltpu.sync_copy(buf0.at[pl.ds(0, TAIL)], acc_sh.at[tb.at[0]], add=True)

    plsc.subcore_barrier()

    @pl.when(sid == 0)
    def _():
        pltpu.sync_copy(acc_sh, out_hbm.at[cid])


_sc_edge = functools.partial(
    pl.kernel,
    mesh=plsc.VectorSubcoreMesh(core_axis_name="c", subcore_axis_name="s"),
    out_type=jax.ShapeDtypeStruct((NC, N, ROWW), jnp.float32),
    compiler_params=pltpu.CompilerParams(
        needs_layout_passes=False, use_tc_tiling_on_sc=False),
    scratch_types=[
        pltpu.VMEM((N,), jnp.float32),
        pltpu.VMEM((N,), jnp.float32),
        pltpu.VMEM((K, ROWW), jnp.float32),
        pltpu.VMEM((K, ROWW), jnp.float32),
        pltpu.VMEM((3, K), jnp.int32),
        pltpu.VMEM((3, K), jnp.int32),
        pltpu.VMEM((K,), jnp.int32),
        pltpu.VMEM((K,), jnp.int32),
        pltpu.VMEM((3, TAIL), jnp.int32),
        pltpu.VMEM((K,), jnp.float32),
        pltpu.VMEM_SHARED((N, ROWW), jnp.float32),
        pltpu.SemaphoreType.DMA,
        pltpu.SemaphoreType.DMA,
        pltpu.SemaphoreType.DMA,
    ],
)(_sc_edge_body)


def _tc_final_body(acc_ref, o_ref):
    a = acc_ref[0] + acc_ref[1]
    col = lax.broadcasted_iota(jnp.int32, (1, ROWW), 1)
    dmask = (col == OUT).astype(jnp.float32)
    denom = jnp.sum(a * dmask, axis=1, keepdims=True)
    s = a[:, 0:OUT]
    o_ref[...] = jnp.tanh(s / jnp.where(denom > 0, denom, 1.0))


_tc_final = pl.pallas_call(
    _tc_final_body,
    out_shape=jax.ShapeDtypeStruct((N, OUT), jnp.float32),
)


def kernel(x, edge_index, adj_vals, kernel, kernel1, kernel2):
    mappedP, c1, c2 = _tc_prep(x, kernel, kernel1, kernel2)
    c1 = c1.reshape(N)
    c2 = c2.reshape(N)
    adj_bits = jax.lax.bitcast_convert_type(adj_vals, jnp.int32)
    e3 = jnp.concatenate([edge_index, adj_bits[None, :]], axis=0)
    acc = _sc_edge(mappedP, e3, c1, c2)
    return _tc_final(acc)
